# baseline (speedup 1.0000x reference)
import functools

import jax
import jax.numpy as jnp
from jax import lax
from jax.experimental import pallas as pl
from jax.experimental.pallas import tpu as pltpu
from jax.experimental.pallas import tpu_sc as plsc

B = 16384
D = 64
K = 5
KP = 8
NC = 2
NS = 16
NW = NC * NS
BPW = B // NW
CH = 128
NCH = BPW // CH
L = 16

_mesh = plsc.VectorSubcoreMesh(core_axis_name="c", subcore_axis_name="s")


@functools.partial(
    pl.kernel,
    mesh=_mesh,
    compiler_params=pltpu.CompilerParams(use_tc_tiling_on_sc=False),
    out_type=jax.ShapeDtypeStruct((B, D), jnp.float32),
    scratch_types=[
        pltpu.VMEM((NCH, CH), jnp.int32),
        pltpu.VMEM((NCH, CH), jnp.int32),
        pltpu.VMEM((BPW, D), jnp.float32),
        pltpu.VMEM((BPW, D), jnp.float32),
        pltpu.SemaphoreType.DMA,
    ],
)
def _sc_gather_mul(user3, product3, uf_hbm, pf_hbm, out_hbm,
                   u_idx, p_idx, u_rows, p_rows, sem):
    wid = lax.axis_index("s") * NC + lax.axis_index("c")
    base = wid * BPW

    pltpu.sync_copy(user3.at[wid], u_idx)
    pltpu.sync_copy(product3.at[wid], p_idx)
    copies = []
    for i in range(NCH):
        copies.append(pltpu.async_copy(
            uf_hbm.at[u_idx.at[i]], u_rows.at[pl.ds(i * CH, CH)], sem))
        copies.append(pltpu.async_copy(
            pf_hbm.at[p_idx.at[i]], p_rows.at[pl.ds(i * CH, CH)], sem))
    for c in copies:
        c.wait()

    def row_body(r, carry):
        for c in range(D // L):
            sl = pl.ds(c * L, L)
            u_rows[r, sl] = u_rows[r, sl] * p_rows[r, sl]
        return carry

    lax.fori_loop(0, BPW, row_body, 0)
    pltpu.sync_copy(u_rows, out_hbm.at[pl.ds(base, BPW)])


def _tc_body(x_ref, w_ref, b_ref, o_ref):
    o_ref[...] = (
        jnp.dot(x_ref[...], w_ref[...], preferred_element_type=jnp.float32)
        + b_ref[...]
    )


_ROWS_BLK = 2048

_tc_logits = pl.pallas_call(
    _tc_body,
    grid=(B // _ROWS_BLK,),
    in_specs=[
        pl.BlockSpec((_ROWS_BLK, D), lambda i: (i, 0)),
        pl.BlockSpec((D, KP), lambda i: (0, 0)),
        pl.BlockSpec((1, KP), lambda i: (0, 0)),
    ],
    out_specs=pl.BlockSpec((_ROWS_BLK, KP), lambda i: (i, 0)),
    out_shape=jax.ShapeDtypeStruct((B, KP), jnp.float32),
)


def kernel(user, product, user_factors, product_factors, W, b):
    user3 = user.astype(jnp.int32).reshape(NW, NCH, CH)
    product3 = product.astype(jnp.int32).reshape(NW, NCH, CH)
    inter = _sc_gather_mul(user3, product3, user_factors, product_factors)
    wt = jnp.zeros((D, KP), jnp.float32).at[:, :K].set(W.T)
    bp = jnp.zeros((1, KP), jnp.float32).at[0, :K].set(b)
    out = _tc_logits(inter, wt, bp)
    return out[:, :K]
